# baseline (speedup 1.0000x reference)
import functools

import jax
import jax.numpy as jnp
from jax import lax
from jax.experimental import pallas as pl
from jax.experimental.pallas import tpu as pltpu
from jax.experimental.pallas import tpu_sc as plsc

N = 10000
E = 320000
D_IN = 128
D_HID = 128
D_OUT = 64

NC = 2
NS = 16
CHUNK = 500
EROWS = E // CHUNK
EROWS_PER_TILE = EROWS // (NC * NS)
ERT_ALL = EROWS // NS
RPT = N // NS
WF = D_OUT // NC

_mesh = plsc.VectorSubcoreMesh(core_axis_name="c", subcore_axis_name="s")
_sc_params = pltpu.CompilerParams(use_tc_tiling_on_sc=False,
                                  needs_layout_passes=False)


def _zero_fill(zb, rows, width):
    @pl.loop(0, rows)
    def _(i):
        for t in range(width // 16):
            zb[i, pl.ds(t * 16, 16)] = jnp.zeros((16,), jnp.float32)


@functools.partial(
    pl.kernel,
    out_type=jax.ShapeDtypeStruct((NC, N, 16), jnp.float32),
    mesh=_mesh,
    scratch_types=[
        pltpu.VMEM((EROWS_PER_TILE, CHUNK), jnp.int32),
        pltpu.VMEM((512,), jnp.float32),
        pltpu.VMEM((2000,), jnp.float32),
        pltpu.VMEM((1000,), jnp.float32),
        pltpu.VMEM((1000, 16), jnp.float32),
        pltpu.VMEM_SHARED((N,), jnp.float32),
    ],
    compiler_params=_sc_params,
)
def _sc_degree(dst_hbm, cnt_hbm, idx_d, ones_v, z1, tmp, rep, acc):
    c = lax.axis_index("c")
    s = lax.axis_index("s")

    @pl.loop(0, 32)
    def _(i):
        ones_v[pl.ds(i * 16, 16)] = jnp.full((16,), 1.0, jnp.float32)

    @pl.loop(0, 125)
    def _(i):
        z1[pl.ds(i * 16, 16)] = jnp.zeros((16,), jnp.float32)

    @pl.when(s == 0)
    def _():
        for k in range(5):
            pltpu.sync_copy(z1, acc.at[pl.ds(k * 2000, 2000)])

    row0 = (c * NS + s) * EROWS_PER_TILE
    pltpu.sync_copy(dst_hbm.at[pl.ds(row0, EROWS_PER_TILE)], idx_d)
    plsc.subcore_barrier()

    @pl.loop(0, EROWS_PER_TILE)
    def _(j):
        pltpu.sync_copy(ones_v.at[pl.ds(0, CHUNK)], acc.at[idx_d.at[j]],
                        add=True)

    plsc.subcore_barrier()

    @pl.when(s < 10)
    def _():
        pltpu.sync_copy(acc.at[pl.ds(s * 1000, 1000)], tmp)

        @pl.loop(0, 1000)
        def _(r):
            idxv = jnp.zeros((16,), jnp.int32) + r
            rep[r, :] = plsc.load_gather(tmp, [idxv])

        pltpu.sync_copy(rep, cnt_hbm.at[c, pl.ds(s * 1000, 1000)])


def _make_sc_scatter(npass):
    W = 64
    outs = [jax.ShapeDtypeStruct((NC, N, W), jnp.float32) for _ in range(npass)]

    @functools.partial(
        pl.kernel,
        out_type=outs,
        mesh=_mesh,
        scratch_types=[
            pltpu.VMEM((EROWS_PER_TILE, CHUNK), jnp.int32),
            pltpu.VMEM((EROWS_PER_TILE, CHUNK), jnp.int32),
            pltpu.VMEM((CHUNK, W), jnp.float32),
            pltpu.VMEM((CHUNK, W), jnp.float32),
            pltpu.VMEM_SHARED((N, W), jnp.float32),
            pltpu.SemaphoreType.DMA,
            pltpu.SemaphoreType.DMA,
        ],
        compiler_params=_sc_params,
    )
    def _scat(src_hbm, dst_hbm, y_hbm, *rest):
        out_refs = rest[:npass]
        idx_s, idx_d, buf_a, buf_b, acc, sem_a, sem_b = rest[npass:]
        c = lax.axis_index("c")
        s = lax.axis_index("s")

        row0 = (c * NS + s) * EROWS_PER_TILE
        pltpu.sync_copy(src_hbm.at[pl.ds(row0, EROWS_PER_TILE)], idx_s)
        pltpu.sync_copy(dst_hbm.at[pl.ds(row0, EROWS_PER_TILE)], idx_d)

        for h in range(npass):
            tab = y_hbm.at[h]
            _zero_fill(buf_a, CHUNK, W)
            pltpu.sync_copy(buf_a, acc.at[pl.ds(s * RPT, CHUNK)])
            pltpu.sync_copy(buf_a.at[pl.ds(0, RPT - CHUNK)],
                            acc.at[pl.ds(s * RPT + CHUNK, RPT - CHUNK)])
            plsc.subcore_barrier()

            pltpu.async_copy(tab.at[idx_s.at[0]], buf_a, sem_a)

            @pl.loop(0, EROWS_PER_TILE - 2, step=2)
            def _(j):
                pltpu.async_copy(tab.at[idx_s.at[j + 1]], buf_b, sem_b)
                pltpu.make_async_copy(tab.at[idx_s.at[j]], buf_a, sem_a).wait()
                pltpu.sync_copy(buf_a, acc.at[idx_d.at[j]], add=True)
                pltpu.async_copy(tab.at[idx_s.at[j + 2]], buf_a, sem_a)
                pltpu.make_async_copy(tab.at[idx_s.at[j + 1]], buf_b,
                                      sem_b).wait()
                pltpu.sync_copy(buf_b, acc.at[idx_d.at[j + 1]], add=True)

            last = EROWS_PER_TILE - 2
            pltpu.async_copy(tab.at[idx_s.at[last + 1]], buf_b, sem_b)
            pltpu.make_async_copy(tab.at[idx_s.at[last]], buf_a, sem_a).wait()
            pltpu.sync_copy(buf_a, acc.at[idx_d.at[last]], add=True)
            pltpu.make_async_copy(tab.at[idx_s.at[last + 1]], buf_b,
                                  sem_b).wait()
            pltpu.sync_copy(buf_b, acc.at[idx_d.at[last + 1]], add=True)

            plsc.subcore_barrier()
            pltpu.sync_copy(acc.at[pl.ds(s * RPT, RPT)],
                            out_refs[h].at[c, pl.ds(s * RPT, RPT)])
            plsc.subcore_barrier()

    return _scat


_sc_scatter_l1 = _make_sc_scatter(2)


@functools.partial(
    pl.kernel,
    out_type=jax.ShapeDtypeStruct((NC, N, WF), jnp.float32),
    mesh=_mesh,
    scratch_types=[
        pltpu.VMEM((ERT_ALL, CHUNK), jnp.int32),
        pltpu.VMEM((ERT_ALL, CHUNK), jnp.int32),
        pltpu.VMEM((CHUNK, WF), jnp.float32),
        pltpu.VMEM((CHUNK, WF), jnp.float32),
        pltpu.VMEM((CHUNK, 16), jnp.float32),
        pltpu.VMEM((WF,), jnp.float32),
        pltpu.VMEM_SHARED((N, WF), jnp.float32),
        pltpu.SemaphoreType.DMA,
        pltpu.SemaphoreType.DMA,
    ],
    compiler_params=_sc_params,
)
def _sc_scatter_l2f(src_hbm, dst_hbm, y2_hbm, dinv_hbm, b2_hbm, out_ref,
                    idx_s, idx_d, buf_a, buf_b, dinv_t, b2_t, acc,
                    sem_a, sem_b):
    c = lax.axis_index("c")
    s = lax.axis_index("s")

    row0 = s * ERT_ALL
    pltpu.sync_copy(src_hbm.at[pl.ds(row0, ERT_ALL)], idx_s)
    pltpu.sync_copy(dst_hbm.at[pl.ds(row0, ERT_ALL)], idx_d)
    pltpu.sync_copy(b2_hbm.at[0, pl.ds(c * WF, WF)], b2_t)

    tab = y2_hbm.at[c]
    _zero_fill(buf_a, CHUNK, WF)
    pltpu.sync_copy(buf_a, acc.at[pl.ds(s * RPT, CHUNK)])
    pltpu.sync_copy(buf_a.at[pl.ds(0, RPT - CHUNK)],
                    acc.at[pl.ds(s * RPT + CHUNK, RPT - CHUNK)])
    plsc.subcore_barrier()

    pltpu.async_copy(tab.at[idx_s.at[0]], buf_a, sem_a)

    @pl.loop(0, ERT_ALL - 2, step=2)
    def _(j):
        pltpu.async_copy(tab.at[idx_s.at[j + 1]], buf_b, sem_b)
        pltpu.make_async_copy(tab.at[idx_s.at[j]], buf_a, sem_a).wait()
        pltpu.sync_copy(buf_a, acc.at[idx_d.at[j]], add=True)
        pltpu.async_copy(tab.at[idx_s.at[j + 2]], buf_a, sem_a)
        pltpu.make_async_copy(tab.at[idx_s.at[j + 1]], buf_b, sem_b).wait()
        pltpu.sync_copy(buf_b, acc.at[idx_d.at[j + 1]], add=True)

    last = ERT_ALL - 2
    pltpu.async_copy(tab.at[idx_s.at[last + 1]], buf_b, sem_b)
    pltpu.make_async_copy(tab.at[idx_s.at[last]], buf_a, sem_a).wait()
    pltpu.sync_copy(buf_a, acc.at[idx_d.at[last]], add=True)
    pltpu.make_async_copy(tab.at[idx_s.at[last + 1]], buf_b, sem_b).wait()
    pltpu.sync_copy(buf_b, acc.at[idx_d.at[last + 1]], add=True)

    plsc.subcore_barrier()

    for off, nr in ((0, CHUNK), (CHUNK, RPT - CHUNK)):
        rows0 = s * RPT + off
        pltpu.sync_copy(acc.at[pl.ds(rows0, nr)], buf_a.at[pl.ds(0, nr)])
        pltpu.sync_copy(y2_hbm.at[c, pl.ds(rows0, nr)], buf_b.at[pl.ds(0, nr)])
        pltpu.sync_copy(dinv_hbm.at[pl.ds(rows0, nr)], dinv_t.at[pl.ds(0, nr)])

        @pl.loop(0, nr)
        def _(r):
            d = dinv_t[r, pl.ds(0, 16)]
            for t in range(WF // 16):
                col = pl.ds(t * 16, 16)
                buf_a[r, col] = (d * (buf_a[r, col] + buf_b[r, col])
                                 + b2_t[pl.ds(t * 16, 16)])

        pltpu.sync_copy(buf_a.at[pl.ds(0, nr)],
                        out_ref.at[c, pl.ds(rows0, nr)])

_B = 1000


def _dinv_from_cnt(cnt_blk):
    deg = cnt_blk[0, :, 0:1] + cnt_blk[1, :, 0:1] + 1.0
    return lax.rsqrt(deg)


def _tc1_body(x_ref, w1_ref, cnt_ref, y1_ref, dinv_ref):
    dinv = _dinv_from_cnt(cnt_ref)
    xw = jnp.dot(x_ref[...], w1_ref[...], preferred_element_type=jnp.float32)
    y1 = xw * dinv
    y1_ref[0] = y1[:, :64]
    y1_ref[1] = y1[:, 64:]
    dinv_ref[...] = jnp.broadcast_to(dinv, (dinv.shape[0], 16))


def _tc2_body(a0_ref, a1_ref, y1_ref, cnt_ref, b1_ref, w2_ref, y2_ref):
    dinv = _dinv_from_cnt(cnt_ref)
    h0 = (a0_ref[0] + a0_ref[1] + y1_ref[0]) * dinv
    h1 = (a1_ref[0] + a1_ref[1] + y1_ref[1]) * dinv
    pre = jnp.concatenate([h0, h1], axis=1) + b1_ref[...]
    h = jnp.maximum(pre, 0.0)
    hw = jnp.dot(h, w2_ref[...], preferred_element_type=jnp.float32)
    y2 = hw * dinv
    y2_ref[0] = y2[:, :WF]
    y2_ref[1] = y2[:, WF:]


def _tc1(x, W1, cnt):
    return pl.pallas_call(
        _tc1_body,
        grid=(N // _B,),
        in_specs=[
            pl.BlockSpec((_B, D_IN), lambda i: (i, 0)),
            pl.BlockSpec((D_IN, D_HID), lambda i: (0, 0)),
            pl.BlockSpec((NC, _B, 16), lambda i: (0, i, 0)),
        ],
        out_specs=[
            pl.BlockSpec((2, _B, 64), lambda i: (0, i, 0)),
            pl.BlockSpec((_B, 16), lambda i: (i, 0)),
        ],
        out_shape=[
            jax.ShapeDtypeStruct((2, N, 64), jnp.float32),
            jax.ShapeDtypeStruct((N, 16), jnp.float32),
        ],
    )(x, W1, cnt)


def _tc2(a0, a1, y1, cnt, b1, W2):
    return pl.pallas_call(
        _tc2_body,
        grid=(N // _B,),
        in_specs=[
            pl.BlockSpec((NC, _B, 64), lambda i: (0, i, 0)),
            pl.BlockSpec((NC, _B, 64), lambda i: (0, i, 0)),
            pl.BlockSpec((2, _B, 64), lambda i: (0, i, 0)),
            pl.BlockSpec((NC, _B, 16), lambda i: (0, i, 0)),
            pl.BlockSpec((1, D_HID), lambda i: (0, 0)),
            pl.BlockSpec((D_HID, D_OUT), lambda i: (0, 0)),
        ],
        out_specs=pl.BlockSpec((2, _B, WF), lambda i: (0, i, 0)),
        out_shape=jax.ShapeDtypeStruct((2, N, WF), jnp.float32),
    )(a0, a1, y1, cnt, b1, W2)


def kernel(x, edge_index, W1, b1, W2, b2):
    src = edge_index[0].astype(jnp.int32).reshape(EROWS, CHUNK)
    dst = edge_index[1].astype(jnp.int32).reshape(EROWS, CHUNK)
    cnt = _sc_degree(dst)
    y1, dinv = _tc1(x, W1, cnt)
    a0, a1 = _sc_scatter_l1(src, dst, y1)
    y2 = _tc2(a0, a1, y1, cnt, b1.reshape(1, D_HID), W2)
    o = _sc_scatter_l2f(src, dst, y2, dinv, b2.reshape(1, D_OUT))
    return jnp.concatenate([o[0], o[1]], axis=1)


# speedup vs baseline: 1.0079x; 1.0079x over previous
import functools

import jax
import jax.numpy as jnp
from jax import lax
from jax.experimental import pallas as pl
from jax.experimental.pallas import tpu as pltpu
from jax.experimental.pallas import tpu_sc as plsc

N = 10000
E = 320000
D_IN = 128
D_HID = 128
D_OUT = 64

NC = 2
NS = 16
CHUNK = 500
EROWS = E // CHUNK
EROWS_PER_TILE = EROWS // (NC * NS)
ERT_ALL = EROWS // NS
RPT = N // NS
WF = D_OUT // NC

_mesh = plsc.VectorSubcoreMesh(core_axis_name="c", subcore_axis_name="s")
_sc_params = pltpu.CompilerParams(use_tc_tiling_on_sc=False,
                                  needs_layout_passes=False)


def _zero_fill(zb, rows, width):
    @pl.loop(0, rows)
    def _(i):
        for t in range(width // 16):
            zb[i, pl.ds(t * 16, 16)] = jnp.zeros((16,), jnp.float32)


@functools.partial(
    pl.kernel,
    out_type=jax.ShapeDtypeStruct((NC, N, 16), jnp.float32),
    mesh=_mesh,
    scratch_types=[
        pltpu.VMEM((EROWS_PER_TILE, CHUNK), jnp.int32),
        pltpu.VMEM((512,), jnp.float32),
        pltpu.VMEM((2000,), jnp.float32),
        pltpu.VMEM((1000,), jnp.float32),
        pltpu.VMEM((1000, 16), jnp.float32),
        pltpu.VMEM_SHARED((N,), jnp.float32),
    ],
    compiler_params=_sc_params,
)
def _sc_degree(dst_hbm, cnt_hbm, idx_d, ones_v, z1, tmp, rep, acc):
    c = lax.axis_index("c")
    s = lax.axis_index("s")

    @pl.loop(0, 32)
    def _(i):
        ones_v[pl.ds(i * 16, 16)] = jnp.full((16,), 1.0, jnp.float32)

    @pl.loop(0, 125)
    def _(i):
        z1[pl.ds(i * 16, 16)] = jnp.zeros((16,), jnp.float32)

    @pl.when(s == 0)
    def _():
        for k in range(5):
            pltpu.sync_copy(z1, acc.at[pl.ds(k * 2000, 2000)])

    row0 = (c * NS + s) * EROWS_PER_TILE
    pltpu.sync_copy(dst_hbm.at[pl.ds(row0, EROWS_PER_TILE)], idx_d)
    plsc.subcore_barrier()

    @pl.loop(0, EROWS_PER_TILE)
    def _(j):
        pltpu.sync_copy(ones_v.at[pl.ds(0, CHUNK)], acc.at[idx_d.at[j]],
                        add=True)

    plsc.subcore_barrier()

    @pl.when(s < 10)
    def _():
        pltpu.sync_copy(acc.at[pl.ds(s * 1000, 1000)], tmp)

        @pl.loop(0, 1000)
        def _(r):
            idxv = jnp.zeros((16,), jnp.int32) + r
            rep[r, :] = plsc.load_gather(tmp, [idxv])

        pltpu.sync_copy(rep, cnt_hbm.at[c, pl.ds(s * 1000, 1000)])


def _make_sc_scatter(npass):
    W = 64
    outs = [jax.ShapeDtypeStruct((NC, N, W), jnp.float32) for _ in range(npass)]

    @functools.partial(
        pl.kernel,
        out_type=outs,
        mesh=_mesh,
        scratch_types=[
            pltpu.VMEM((EROWS_PER_TILE, CHUNK), jnp.int32),
            pltpu.VMEM((EROWS_PER_TILE, CHUNK), jnp.int32),
            pltpu.VMEM((CHUNK, W), jnp.float32),
            pltpu.VMEM((CHUNK, W), jnp.float32),
            pltpu.VMEM_SHARED((N, W), jnp.float32),
            pltpu.SemaphoreType.DMA,
            pltpu.SemaphoreType.DMA,
        ],
        compiler_params=_sc_params,
    )
    def _scat(src_hbm, dst_hbm, y_hbm, *rest):
        out_refs = rest[:npass]
        idx_s, idx_d, buf_a, buf_b, acc, sem_a, sem_b = rest[npass:]
        c = lax.axis_index("c")
        s = lax.axis_index("s")

        row0 = (c * NS + s) * EROWS_PER_TILE
        pltpu.sync_copy(src_hbm.at[pl.ds(row0, EROWS_PER_TILE)], idx_s)
        pltpu.sync_copy(dst_hbm.at[pl.ds(row0, EROWS_PER_TILE)], idx_d)

        for h in range(npass):
            tab = y_hbm.at[h]
            _zero_fill(buf_a, CHUNK, W)
            pltpu.sync_copy(buf_a, acc.at[pl.ds(s * RPT, CHUNK)])
            pltpu.sync_copy(buf_a.at[pl.ds(0, RPT - CHUNK)],
                            acc.at[pl.ds(s * RPT + CHUNK, RPT - CHUNK)])
            plsc.subcore_barrier()

            pltpu.async_copy(tab.at[idx_s.at[0]], buf_a, sem_a)

            @pl.loop(0, EROWS_PER_TILE - 2, step=2)
            def _(j):
                pltpu.async_copy(tab.at[idx_s.at[j + 1]], buf_b, sem_b)
                pltpu.make_async_copy(tab.at[idx_s.at[j]], buf_a, sem_a).wait()
                pltpu.sync_copy(buf_a, acc.at[idx_d.at[j]], add=True)
                pltpu.async_copy(tab.at[idx_s.at[j + 2]], buf_a, sem_a)
                pltpu.make_async_copy(tab.at[idx_s.at[j + 1]], buf_b,
                                      sem_b).wait()
                pltpu.sync_copy(buf_b, acc.at[idx_d.at[j + 1]], add=True)

            last = EROWS_PER_TILE - 2
            pltpu.async_copy(tab.at[idx_s.at[last + 1]], buf_b, sem_b)
            pltpu.make_async_copy(tab.at[idx_s.at[last]], buf_a, sem_a).wait()
            pltpu.sync_copy(buf_a, acc.at[idx_d.at[last]], add=True)
            pltpu.make_async_copy(tab.at[idx_s.at[last + 1]], buf_b,
                                  sem_b).wait()
            pltpu.sync_copy(buf_b, acc.at[idx_d.at[last + 1]], add=True)

            plsc.subcore_barrier()
            pltpu.sync_copy(acc.at[pl.ds(s * RPT, RPT)],
                            out_refs[h].at[c, pl.ds(s * RPT, RPT)])
            plsc.subcore_barrier()

    return _scat


_sc_scatter_l1 = _make_sc_scatter(2)


@functools.partial(
    pl.kernel,
    out_type=jax.ShapeDtypeStruct((NC, N, WF), jnp.float32),
    mesh=_mesh,
    scratch_types=[
        pltpu.VMEM((ERT_ALL, CHUNK), jnp.int32),
        pltpu.VMEM((ERT_ALL, CHUNK), jnp.int32),
        pltpu.VMEM((CHUNK, WF), jnp.float32),
        pltpu.VMEM((CHUNK, WF), jnp.float32),
        pltpu.VMEM((CHUNK, 16), jnp.float32),
        pltpu.VMEM((WF,), jnp.float32),
        pltpu.VMEM_SHARED((N, WF), jnp.float32),
        pltpu.SemaphoreType.DMA,
        pltpu.SemaphoreType.DMA,
    ],
    compiler_params=_sc_params,
)
def _sc_scatter_l2f(src_hbm, dst_hbm, y2_hbm, dinv_hbm, b2_hbm, out_ref,
                    idx_s, idx_d, buf_a, buf_b, dinv_t, b2_t, acc,
                    sem_a, sem_b):
    c = lax.axis_index("c")
    s = lax.axis_index("s")

    row0 = s * ERT_ALL
    pltpu.sync_copy(src_hbm.at[pl.ds(row0, ERT_ALL)], idx_s)
    pltpu.sync_copy(dst_hbm.at[pl.ds(row0, ERT_ALL)], idx_d)
    pltpu.sync_copy(b2_hbm.at[0, pl.ds(c * WF, WF)], b2_t)

    tab = y2_hbm.at[c]
    _zero_fill(buf_a, CHUNK, WF)
    pltpu.sync_copy(buf_a, acc.at[pl.ds(s * RPT, CHUNK)])
    pltpu.sync_copy(buf_a.at[pl.ds(0, RPT - CHUNK)],
                    acc.at[pl.ds(s * RPT + CHUNK, RPT - CHUNK)])
    plsc.subcore_barrier()

    pltpu.async_copy(tab.at[idx_s.at[0]], buf_a, sem_a)

    @pl.loop(0, ERT_ALL - 2, step=2)
    def _(j):
        pltpu.async_copy(tab.at[idx_s.at[j + 1]], buf_b, sem_b)
        pltpu.make_async_copy(tab.at[idx_s.at[j]], buf_a, sem_a).wait()
        pltpu.sync_copy(buf_a, acc.at[idx_d.at[j]], add=True)
        pltpu.async_copy(tab.at[idx_s.at[j + 2]], buf_a, sem_a)
        pltpu.make_async_copy(tab.at[idx_s.at[j + 1]], buf_b, sem_b).wait()
        pltpu.sync_copy(buf_b, acc.at[idx_d.at[j + 1]], add=True)

    last = ERT_ALL - 2
    pltpu.async_copy(tab.at[idx_s.at[last + 1]], buf_b, sem_b)
    pltpu.make_async_copy(tab.at[idx_s.at[last]], buf_a, sem_a).wait()
    pltpu.sync_copy(buf_a, acc.at[idx_d.at[last]], add=True)
    pltpu.make_async_copy(tab.at[idx_s.at[last + 1]], buf_b, sem_b).wait()
    pltpu.sync_copy(buf_b, acc.at[idx_d.at[last + 1]], add=True)

    plsc.subcore_barrier()

    for off, nr in ((0, CHUNK), (CHUNK, RPT - CHUNK)):
        rows0 = s * RPT + off
        pltpu.sync_copy(acc.at[pl.ds(rows0, nr)], buf_a.at[pl.ds(0, nr)])
        pltpu.sync_copy(y2_hbm.at[c, pl.ds(rows0, nr)], buf_b.at[pl.ds(0, nr)])
        pltpu.sync_copy(dinv_hbm.at[pl.ds(rows0, nr)], dinv_t.at[pl.ds(0, nr)])

        @pl.loop(0, nr)
        def _(r):
            d = dinv_t[r, pl.ds(0, 16)]
            for t in range(WF // 16):
                col = pl.ds(t * 16, 16)
                buf_a[r, col] = (d * (buf_a[r, col] + buf_b[r, col])
                                 + b2_t[pl.ds(t * 16, 16)])

        pltpu.sync_copy(buf_a.at[pl.ds(0, nr)],
                        out_ref.at[c, pl.ds(rows0, nr)])

_B = 2000


def _dinv_from_cnt(cnt_blk):
    deg = cnt_blk[0, :, 0:1] + cnt_blk[1, :, 0:1] + 1.0
    return lax.rsqrt(deg)


def _tc1_body(x_ref, w1_ref, cnt_ref, y1_ref, dinv_ref):
    dinv = _dinv_from_cnt(cnt_ref)
    xw = jnp.dot(x_ref[...], w1_ref[...], preferred_element_type=jnp.float32)
    y1 = xw * dinv
    y1_ref[0] = y1[:, :64]
    y1_ref[1] = y1[:, 64:]
    dinv_ref[...] = jnp.broadcast_to(dinv, (dinv.shape[0], 16))


def _tc2_body(a0_ref, a1_ref, y1_ref, cnt_ref, b1_ref, w2_ref, y2_ref):
    dinv = _dinv_from_cnt(cnt_ref)
    h0 = (a0_ref[0] + a0_ref[1] + y1_ref[0]) * dinv
    h1 = (a1_ref[0] + a1_ref[1] + y1_ref[1]) * dinv
    pre = jnp.concatenate([h0, h1], axis=1) + b1_ref[...]
    h = jnp.maximum(pre, 0.0)
    hw = jnp.dot(h, w2_ref[...], preferred_element_type=jnp.float32)
    y2 = hw * dinv
    y2_ref[0] = y2[:, :WF]
    y2_ref[1] = y2[:, WF:]


def _tc1(x, W1, cnt):
    return pl.pallas_call(
        _tc1_body,
        grid=(N // _B,),
        in_specs=[
            pl.BlockSpec((_B, D_IN), lambda i: (i, 0)),
            pl.BlockSpec((D_IN, D_HID), lambda i: (0, 0)),
            pl.BlockSpec((NC, _B, 16), lambda i: (0, i, 0)),
        ],
        out_specs=[
            pl.BlockSpec((2, _B, 64), lambda i: (0, i, 0)),
            pl.BlockSpec((_B, 16), lambda i: (i, 0)),
        ],
        out_shape=[
            jax.ShapeDtypeStruct((2, N, 64), jnp.float32),
            jax.ShapeDtypeStruct((N, 16), jnp.float32),
        ],
    )(x, W1, cnt)


def _tc2(a0, a1, y1, cnt, b1, W2):
    return pl.pallas_call(
        _tc2_body,
        grid=(N // _B,),
        in_specs=[
            pl.BlockSpec((NC, _B, 64), lambda i: (0, i, 0)),
            pl.BlockSpec((NC, _B, 64), lambda i: (0, i, 0)),
            pl.BlockSpec((2, _B, 64), lambda i: (0, i, 0)),
            pl.BlockSpec((NC, _B, 16), lambda i: (0, i, 0)),
            pl.BlockSpec((1, D_HID), lambda i: (0, 0)),
            pl.BlockSpec((D_HID, D_OUT), lambda i: (0, 0)),
        ],
        out_specs=pl.BlockSpec((2, _B, WF), lambda i: (0, i, 0)),
        out_shape=jax.ShapeDtypeStruct((2, N, WF), jnp.float32),
    )(a0, a1, y1, cnt, b1, W2)


def kernel(x, edge_index, W1, b1, W2, b2):
    src = edge_index[0].astype(jnp.int32).reshape(EROWS, CHUNK)
    dst = edge_index[1].astype(jnp.int32).reshape(EROWS, CHUNK)
    cnt = _sc_degree(dst)
    y1, dinv = _tc1(x, W1, cnt)
    a0, a1 = _sc_scatter_l1(src, dst, y1)
    y2 = _tc2(a0, a1, y1, cnt, b1.reshape(1, D_HID), W2)
    o = _sc_scatter_l2f(src, dst, y2, dinv, b2.reshape(1, D_OUT))
    return jnp.concatenate([o[0], o[1]], axis=1)


# speedup vs baseline: 1.1298x; 1.1209x over previous
import functools

import jax
import jax.numpy as jnp
from jax import lax
from jax.experimental import pallas as pl
from jax.experimental.pallas import tpu as pltpu
from jax.experimental.pallas import tpu_sc as plsc

N = 10000
E = 320000
D_IN = 128
D_HID = 128
D_OUT = 64

NC = 2
NS = 16
CHUNK = 500
EROWS = E // CHUNK
EROWS_PER_TILE = EROWS // (NC * NS)
ERT_ALL = EROWS // NS
RPT = N // NS
WF = D_OUT // NC

_mesh = plsc.VectorSubcoreMesh(core_axis_name="c", subcore_axis_name="s")
_sc_params = pltpu.CompilerParams(use_tc_tiling_on_sc=False,
                                  needs_layout_passes=False)


def _zero_fill(zb, rows, width):
    @pl.loop(0, rows)
    def _(i):
        for t in range(width // 16):
            zb[i, pl.ds(t * 16, 16)] = jnp.zeros((16,), jnp.float32)


@functools.partial(
    pl.kernel,
    out_type=jax.ShapeDtypeStruct((NC, N, 16), jnp.float32),
    mesh=_mesh,
    scratch_types=[
        pltpu.VMEM((EROWS_PER_TILE, CHUNK), jnp.int32),
        pltpu.VMEM((512,), jnp.float32),
        pltpu.VMEM((2000,), jnp.float32),
        pltpu.VMEM((1000,), jnp.float32),
        pltpu.VMEM((1000, 16), jnp.float32),
        pltpu.VMEM_SHARED((N,), jnp.float32),
    ],
    compiler_params=_sc_params,
)
def _sc_degree(dst_hbm, cnt_hbm, idx_d, ones_v, z1, tmp, rep, acc):
    c = lax.axis_index("c")
    s = lax.axis_index("s")

    @pl.loop(0, 32)
    def _(i):
        ones_v[pl.ds(i * 16, 16)] = jnp.full((16,), 1.0, jnp.float32)

    @pl.loop(0, 125)
    def _(i):
        z1[pl.ds(i * 16, 16)] = jnp.zeros((16,), jnp.float32)

    @pl.when(s == 0)
    def _():
        for k in range(5):
            pltpu.sync_copy(z1, acc.at[pl.ds(k * 2000, 2000)])

    row0 = (c * NS + s) * EROWS_PER_TILE
    pltpu.sync_copy(dst_hbm.at[pl.ds(row0, EROWS_PER_TILE)], idx_d)
    plsc.subcore_barrier()

    @pl.loop(0, EROWS_PER_TILE)
    def _(j):
        pltpu.sync_copy(ones_v.at[pl.ds(0, CHUNK)], acc.at[idx_d.at[j]],
                        add=True)

    plsc.subcore_barrier()

    @pl.when(s < 10)
    def _():
        pltpu.sync_copy(acc.at[pl.ds(s * 1000, 1000)], tmp)

        @pl.loop(0, 1000)
        def _(r):
            idxv = jnp.zeros((16,), jnp.int32) + r
            rep[r, :] = plsc.load_gather(tmp, [idxv])

        pltpu.sync_copy(rep, cnt_hbm.at[c, pl.ds(s * 1000, 1000)])


CH1 = 125
ER1 = E // CH1
ER1_PER_TILE = ER1 // (NC * NS)
BR = 16
NBLK = ER1_PER_TILE // BR


@functools.partial(
    pl.kernel,
    out_type=jax.ShapeDtypeStruct((NC, N, D_HID), jnp.float32),
    mesh=_mesh,
    scratch_types=[
        pltpu.VMEM((2, BR, CH1), jnp.int32),
        pltpu.VMEM((2, BR, CH1), jnp.int32),
        pltpu.VMEM((CH1, D_HID), jnp.float32),
        pltpu.VMEM((CH1, D_HID), jnp.float32),
        pltpu.VMEM_SHARED((N, D_HID), jnp.float32),
        pltpu.SemaphoreType.DMA,
        pltpu.SemaphoreType.DMA,
        pltpu.SemaphoreType.DMA,
        pltpu.SemaphoreType.DMA,
    ],
    compiler_params=_sc_params,
)
def _sc_scatter_l1(src_hbm, dst_hbm, y_hbm, out_ref,
                   idx_s, idx_d, buf_a, buf_b, acc,
                   sem_a, sem_b, sem_is, sem_id):
    c = lax.axis_index("c")
    s = lax.axis_index("s")
    row0 = (c * NS + s) * ER1_PER_TILE

    pltpu.sync_copy(src_hbm.at[pl.ds(row0, BR)], idx_s.at[0])
    pltpu.sync_copy(dst_hbm.at[pl.ds(row0, BR)], idx_d.at[0])

    _zero_fill(buf_a, CH1, D_HID)
    for k in range(RPT // CH1):
        pltpu.sync_copy(buf_a, acc.at[pl.ds(s * RPT + k * CH1, CH1)])
    plsc.subcore_barrier()

    pltpu.async_copy(y_hbm.at[idx_s.at[0, 0]], buf_a, sem_a)

    for blk in range(NBLK):
        p = blk % 2
        q = (blk + 1) % 2
        nrow0 = row0 + (blk + 1) * BR
        if blk + 1 < NBLK:
            pltpu.async_copy(src_hbm.at[pl.ds(nrow0, BR)], idx_s.at[q],
                             sem_is)
            pltpu.async_copy(dst_hbm.at[pl.ds(nrow0, BR)], idx_d.at[q],
                             sem_id)

        @pl.loop(0, BR - 2, step=2)
        def _(j):
            pltpu.async_copy(y_hbm.at[idx_s.at[p, j + 1]], buf_b, sem_b)
            pltpu.make_async_copy(y_hbm.at[idx_s.at[p, j]], buf_a,
                                  sem_a).wait()
            pltpu.sync_copy(buf_a, acc.at[idx_d.at[p, j]], add=True)
            pltpu.async_copy(y_hbm.at[idx_s.at[p, j + 2]], buf_a, sem_a)
            pltpu.make_async_copy(y_hbm.at[idx_s.at[p, j + 1]], buf_b,
                                  sem_b).wait()
            pltpu.sync_copy(buf_b, acc.at[idx_d.at[p, j + 1]], add=True)

        last = BR - 2
        pltpu.async_copy(y_hbm.at[idx_s.at[p, last + 1]], buf_b, sem_b)
        pltpu.make_async_copy(y_hbm.at[idx_s.at[p, last]], buf_a,
                              sem_a).wait()
        pltpu.sync_copy(buf_a, acc.at[idx_d.at[p, last]], add=True)
        if blk + 1 < NBLK:
            pltpu.make_async_copy(src_hbm.at[pl.ds(nrow0, BR)], idx_s.at[q],
                                  sem_is).wait()
            pltpu.make_async_copy(dst_hbm.at[pl.ds(nrow0, BR)], idx_d.at[q],
                                  sem_id).wait()
            pltpu.async_copy(y_hbm.at[idx_s.at[q, 0]], buf_a, sem_a)
        pltpu.make_async_copy(y_hbm.at[idx_s.at[p, last + 1]], buf_b,
                              sem_b).wait()
        pltpu.sync_copy(buf_b, acc.at[idx_d.at[p, last + 1]], add=True)

    plsc.subcore_barrier()
    pltpu.sync_copy(acc.at[pl.ds(s * RPT, RPT)],
                    out_ref.at[c, pl.ds(s * RPT, RPT)])


@functools.partial(
    pl.kernel,
    out_type=jax.ShapeDtypeStruct((NC, N, WF), jnp.float32),
    mesh=_mesh,
    scratch_types=[
        pltpu.VMEM((ERT_ALL, CHUNK), jnp.int32),
        pltpu.VMEM((ERT_ALL, CHUNK), jnp.int32),
        pltpu.VMEM((CHUNK, WF), jnp.float32),
        pltpu.VMEM((CHUNK, WF), jnp.float32),
        pltpu.VMEM((CHUNK, 16), jnp.float32),
        pltpu.VMEM((WF,), jnp.float32),
        pltpu.VMEM_SHARED((N, WF), jnp.float32),
        pltpu.SemaphoreType.DMA,
        pltpu.SemaphoreType.DMA,
    ],
    compiler_params=_sc_params,
)
def _sc_scatter_l2f(src_hbm, dst_hbm, y2_hbm, dinv_hbm, b2_hbm, out_ref,
                    idx_s, idx_d, buf_a, buf_b, dinv_t, b2_t, acc,
                    sem_a, sem_b):
    c = lax.axis_index("c")
    s = lax.axis_index("s")

    row0 = s * ERT_ALL
    pltpu.sync_copy(src_hbm.at[pl.ds(row0, ERT_ALL)], idx_s)
    pltpu.sync_copy(dst_hbm.at[pl.ds(row0, ERT_ALL)], idx_d)
    pltpu.sync_copy(b2_hbm.at[0, pl.ds(c * WF, WF)], b2_t)

    tab = y2_hbm.at[c]
    _zero_fill(buf_a, CHUNK, WF)
    pltpu.sync_copy(buf_a, acc.at[pl.ds(s * RPT, CHUNK)])
    pltpu.sync_copy(buf_a.at[pl.ds(0, RPT - CHUNK)],
                    acc.at[pl.ds(s * RPT + CHUNK, RPT - CHUNK)])
    plsc.subcore_barrier()

    pltpu.async_copy(tab.at[idx_s.at[0]], buf_a, sem_a)

    @pl.loop(0, ERT_ALL - 2, step=2)
    def _(j):
        pltpu.async_copy(tab.at[idx_s.at[j + 1]], buf_b, sem_b)
        pltpu.make_async_copy(tab.at[idx_s.at[j]], buf_a, sem_a).wait()
        pltpu.sync_copy(buf_a, acc.at[idx_d.at[j]], add=True)
        pltpu.async_copy(tab.at[idx_s.at[j + 2]], buf_a, sem_a)
        pltpu.make_async_copy(tab.at[idx_s.at[j + 1]], buf_b, sem_b).wait()
        pltpu.sync_copy(buf_b, acc.at[idx_d.at[j + 1]], add=True)

    last = ERT_ALL - 2
    pltpu.async_copy(tab.at[idx_s.at[last + 1]], buf_b, sem_b)
    pltpu.make_async_copy(tab.at[idx_s.at[last]], buf_a, sem_a).wait()
    pltpu.sync_copy(buf_a, acc.at[idx_d.at[last]], add=True)
    pltpu.make_async_copy(tab.at[idx_s.at[last + 1]], buf_b, sem_b).wait()
    pltpu.sync_copy(buf_b, acc.at[idx_d.at[last + 1]], add=True)

    plsc.subcore_barrier()

    for off, nr in ((0, CHUNK), (CHUNK, RPT - CHUNK)):
        rows0 = s * RPT + off
        pltpu.sync_copy(acc.at[pl.ds(rows0, nr)], buf_a.at[pl.ds(0, nr)])
        pltpu.sync_copy(y2_hbm.at[c, pl.ds(rows0, nr)], buf_b.at[pl.ds(0, nr)])
        pltpu.sync_copy(dinv_hbm.at[pl.ds(rows0, nr)], dinv_t.at[pl.ds(0, nr)])

        @pl.loop(0, nr)
        def _(r):
            d = dinv_t[r, pl.ds(0, 16)]
            for t in range(WF // 16):
                col = pl.ds(t * 16, 16)
                buf_a[r, col] = (d * (buf_a[r, col] + buf_b[r, col])
                                 + b2_t[pl.ds(t * 16, 16)])

        pltpu.sync_copy(buf_a.at[pl.ds(0, nr)],
                        out_ref.at[c, pl.ds(rows0, nr)])

_B = 2000


def _dinv_from_cnt(cnt_blk):
    deg = cnt_blk[0, :, 0:1] + cnt_blk[1, :, 0:1] + 1.0
    return lax.rsqrt(deg)


def _tc1_body(x_ref, w1_ref, cnt_ref, y1_ref, dinv_ref):
    dinv = _dinv_from_cnt(cnt_ref)
    xw = jnp.dot(x_ref[...], w1_ref[...], preferred_element_type=jnp.float32)
    y1_ref[...] = xw * dinv
    dinv_ref[...] = jnp.broadcast_to(dinv, (dinv.shape[0], 16))


def _tc2_body(a_ref, y1_ref, cnt_ref, b1_ref, w2_ref, y2_ref):
    dinv = _dinv_from_cnt(cnt_ref)
    pre = (a_ref[0] + a_ref[1] + y1_ref[...]) * dinv + b1_ref[...]
    h = jnp.maximum(pre, 0.0)
    hw = jnp.dot(h, w2_ref[...], preferred_element_type=jnp.float32)
    y2 = hw * dinv
    y2_ref[0] = y2[:, :WF]
    y2_ref[1] = y2[:, WF:]


def _tc1(x, W1, cnt):
    return pl.pallas_call(
        _tc1_body,
        grid=(N // _B,),
        in_specs=[
            pl.BlockSpec((_B, D_IN), lambda i: (i, 0)),
            pl.BlockSpec((D_IN, D_HID), lambda i: (0, 0)),
            pl.BlockSpec((NC, _B, 16), lambda i: (0, i, 0)),
        ],
        out_specs=[
            pl.BlockSpec((_B, D_HID), lambda i: (i, 0)),
            pl.BlockSpec((_B, 16), lambda i: (i, 0)),
        ],
        out_shape=[
            jax.ShapeDtypeStruct((N, D_HID), jnp.float32),
            jax.ShapeDtypeStruct((N, 16), jnp.float32),
        ],
    )(x, W1, cnt)


def _tc2(a, y1, cnt, b1, W2):
    return pl.pallas_call(
        _tc2_body,
        grid=(N // _B,),
        in_specs=[
            pl.BlockSpec((NC, _B, D_HID), lambda i: (0, i, 0)),
            pl.BlockSpec((_B, D_HID), lambda i: (i, 0)),
            pl.BlockSpec((NC, _B, 16), lambda i: (0, i, 0)),
            pl.BlockSpec((1, D_HID), lambda i: (0, 0)),
            pl.BlockSpec((D_HID, D_OUT), lambda i: (0, 0)),
        ],
        out_specs=pl.BlockSpec((2, _B, WF), lambda i: (0, i, 0)),
        out_shape=jax.ShapeDtypeStruct((2, N, WF), jnp.float32),
    )(a, y1, cnt, b1, W2)


def kernel(x, edge_index, W1, b1, W2, b2):
    src = edge_index[0].astype(jnp.int32)
    dst = edge_index[1].astype(jnp.int32)
    src5 = src.reshape(EROWS, CHUNK)
    dst5 = dst.reshape(EROWS, CHUNK)
    src1 = src.reshape(ER1, CH1)
    dst1 = dst.reshape(ER1, CH1)
    cnt = _sc_degree(dst5)
    y1, dinv = _tc1(x, W1, cnt)
    a = _sc_scatter_l1(src1, dst1, y1)
    y2 = _tc2(a, y1, cnt, b1.reshape(1, D_HID), W2)
    o = _sc_scatter_l2f(src5, dst5, y2, dinv, b2.reshape(1, D_OUT))
    return jnp.concatenate([o[0], o[1]], axis=1)


# speedup vs baseline: 1.1739x; 1.0390x over previous
import functools

import jax
import jax.numpy as jnp
from jax import lax
from jax.experimental import pallas as pl
from jax.experimental.pallas import tpu as pltpu
from jax.experimental.pallas import tpu_sc as plsc

N = 10000
E = 320000
D_IN = 128
D_HID = 128
D_OUT = 64

NC = 2
NS = 16
CHUNK = 500
EROWS = E // CHUNK
EROWS_PER_TILE = EROWS // (NC * NS)
ERT_ALL = EROWS // NS
RPT = N // NS
WF = D_OUT // NC

_mesh = plsc.VectorSubcoreMesh(core_axis_name="c", subcore_axis_name="s")
_sc_params = pltpu.CompilerParams(use_tc_tiling_on_sc=False,
                                  needs_layout_passes=False)


def _zero_fill(zb, rows, width):
    @pl.loop(0, rows)
    def _(i):
        for t in range(width // 16):
            zb[i, pl.ds(t * 16, 16)] = jnp.zeros((16,), jnp.float32)


@functools.partial(
    pl.kernel,
    out_type=jax.ShapeDtypeStruct((NC, N, 16), jnp.float32),
    mesh=_mesh,
    scratch_types=[
        pltpu.VMEM((EROWS_PER_TILE, CHUNK), jnp.int32),
        pltpu.VMEM((512,), jnp.float32),
        pltpu.VMEM((2000,), jnp.float32),
        pltpu.VMEM((1000,), jnp.float32),
        pltpu.VMEM((1000, 16), jnp.float32),
        pltpu.VMEM_SHARED((N,), jnp.float32),
    ],
    compiler_params=_sc_params,
)
def _sc_degree(dst_hbm, cnt_hbm, idx_d, ones_v, z1, tmp, rep, acc):
    c = lax.axis_index("c")
    s = lax.axis_index("s")

    @pl.loop(0, 32)
    def _(i):
        ones_v[pl.ds(i * 16, 16)] = jnp.full((16,), 1.0, jnp.float32)

    @pl.loop(0, 125)
    def _(i):
        z1[pl.ds(i * 16, 16)] = jnp.zeros((16,), jnp.float32)

    @pl.when(s == 0)
    def _():
        for k in range(5):
            pltpu.sync_copy(z1, acc.at[pl.ds(k * 2000, 2000)])

    row0 = (c * NS + s) * EROWS_PER_TILE
    pltpu.sync_copy(dst_hbm.at[pl.ds(row0, EROWS_PER_TILE)], idx_d)
    plsc.subcore_barrier()

    @pl.loop(0, EROWS_PER_TILE)
    def _(j):
        pltpu.sync_copy(ones_v.at[pl.ds(0, CHUNK)], acc.at[idx_d.at[j]],
                        add=True)

    plsc.subcore_barrier()

    @pl.when(s < 10)
    def _():
        pltpu.sync_copy(acc.at[pl.ds(s * 1000, 1000)], tmp)

        @pl.loop(0, 1000)
        def _(r):
            idxv = jnp.zeros((16,), jnp.int32) + r
            rep[r, :] = plsc.load_gather(tmp, [idxv])

        pltpu.sync_copy(rep, cnt_hbm.at[c, pl.ds(s * 1000, 1000)])


CH1 = 125
ER1 = E // CH1
ER1_PER_TILE = ER1 // (NC * NS)
BR = 16
NBLK = ER1_PER_TILE // BR


@functools.partial(
    pl.kernel,
    out_type=jax.ShapeDtypeStruct((NC, N, D_HID), jnp.float32),
    mesh=_mesh,
    scratch_types=[
        pltpu.VMEM((2, BR, CH1), jnp.int32),
        pltpu.VMEM((2, BR, CH1), jnp.int32),
        pltpu.VMEM((CH1, D_HID), jnp.float32),
        pltpu.VMEM((CH1, D_HID), jnp.float32),
        pltpu.VMEM_SHARED((N, D_HID), jnp.float32),
        pltpu.SemaphoreType.DMA,
        pltpu.SemaphoreType.DMA,
        pltpu.SemaphoreType.DMA,
        pltpu.SemaphoreType.DMA,
    ],
    compiler_params=_sc_params,
)
def _sc_scatter_l1(src_hbm, dst_hbm, y_hbm, out_ref,
                   idx_s, idx_d, buf_a, buf_b, acc,
                   sem_a, sem_b, sem_is, sem_id):
    c = lax.axis_index("c")
    s = lax.axis_index("s")
    row0 = (c * NS + s) * ER1_PER_TILE

    pltpu.sync_copy(src_hbm.at[pl.ds(row0, BR)], idx_s.at[0])
    pltpu.sync_copy(dst_hbm.at[pl.ds(row0, BR)], idx_d.at[0])

    _zero_fill(buf_a, CH1, D_HID)
    for k in range(RPT // CH1):
        pltpu.sync_copy(buf_a, acc.at[pl.ds(s * RPT + k * CH1, CH1)])
    plsc.subcore_barrier()

    pltpu.async_copy(y_hbm.at[idx_s.at[0, 0]], buf_a, sem_a)

    for blk in range(NBLK):
        p = blk % 2
        q = (blk + 1) % 2
        nrow0 = row0 + (blk + 1) * BR
        if blk + 1 < NBLK:
            pltpu.async_copy(src_hbm.at[pl.ds(nrow0, BR)], idx_s.at[q],
                             sem_is)
            pltpu.async_copy(dst_hbm.at[pl.ds(nrow0, BR)], idx_d.at[q],
                             sem_id)

        @pl.loop(0, BR - 2, step=2)
        def _(j):
            pltpu.async_copy(y_hbm.at[idx_s.at[p, j + 1]], buf_b, sem_b)
            pltpu.make_async_copy(y_hbm.at[idx_s.at[p, j]], buf_a,
                                  sem_a).wait()
            pltpu.sync_copy(buf_a, acc.at[idx_d.at[p, j]], add=True)
            pltpu.async_copy(y_hbm.at[idx_s.at[p, j + 2]], buf_a, sem_a)
            pltpu.make_async_copy(y_hbm.at[idx_s.at[p, j + 1]], buf_b,
                                  sem_b).wait()
            pltpu.sync_copy(buf_b, acc.at[idx_d.at[p, j + 1]], add=True)

        last = BR - 2
        pltpu.async_copy(y_hbm.at[idx_s.at[p, last + 1]], buf_b, sem_b)
        pltpu.make_async_copy(y_hbm.at[idx_s.at[p, last]], buf_a,
                              sem_a).wait()
        pltpu.sync_copy(buf_a, acc.at[idx_d.at[p, last]], add=True)
        if blk + 1 < NBLK:
            pltpu.make_async_copy(src_hbm.at[pl.ds(nrow0, BR)], idx_s.at[q],
                                  sem_is).wait()
            pltpu.make_async_copy(dst_hbm.at[pl.ds(nrow0, BR)], idx_d.at[q],
                                  sem_id).wait()
            pltpu.async_copy(y_hbm.at[idx_s.at[q, 0]], buf_a, sem_a)
        pltpu.make_async_copy(y_hbm.at[idx_s.at[p, last + 1]], buf_b,
                              sem_b).wait()
        pltpu.sync_copy(buf_b, acc.at[idx_d.at[p, last + 1]], add=True)

    plsc.subcore_barrier()
    pltpu.sync_copy(acc.at[pl.ds(s * RPT, RPT)],
                    out_ref.at[c, pl.ds(s * RPT, RPT)])


@functools.partial(
    pl.kernel,
    out_type=jax.ShapeDtypeStruct((N, D_OUT), jnp.float32),
    mesh=_mesh,
    scratch_types=[
        pltpu.VMEM((ERT_ALL, CHUNK), jnp.int32),
        pltpu.VMEM((ERT_ALL, CHUNK), jnp.int32),
        pltpu.VMEM((CHUNK, WF), jnp.float32),
        pltpu.VMEM((CHUNK, WF), jnp.float32),
        pltpu.VMEM((CHUNK, 16), jnp.float32),
        pltpu.VMEM((WF,), jnp.float32),
        pltpu.VMEM_SHARED((N, WF), jnp.float32),
        pltpu.SemaphoreType.DMA,
        pltpu.SemaphoreType.DMA,
    ],
    compiler_params=_sc_params,
)
def _sc_scatter_l2f(src_hbm, dst_hbm, y2_hbm, dinv_hbm, b2_hbm, out_ref,
                    idx_s, idx_d, buf_a, buf_b, dinv_t, b2_t, acc,
                    sem_a, sem_b):
    c = lax.axis_index("c")
    s = lax.axis_index("s")

    row0 = s * ERT_ALL
    pltpu.sync_copy(src_hbm.at[pl.ds(row0, ERT_ALL)], idx_s)
    pltpu.sync_copy(dst_hbm.at[pl.ds(row0, ERT_ALL)], idx_d)
    pltpu.sync_copy(b2_hbm.at[0, pl.ds(c * WF, WF)], b2_t)

    tab = y2_hbm.at[c]
    _zero_fill(buf_a, CHUNK, WF)
    pltpu.sync_copy(buf_a, acc.at[pl.ds(s * RPT, CHUNK)])
    pltpu.sync_copy(buf_a.at[pl.ds(0, RPT - CHUNK)],
                    acc.at[pl.ds(s * RPT + CHUNK, RPT - CHUNK)])
    plsc.subcore_barrier()

    pltpu.async_copy(tab.at[idx_s.at[0]], buf_a, sem_a)

    @pl.loop(0, ERT_ALL - 2, step=2)
    def _(j):
        pltpu.async_copy(tab.at[idx_s.at[j + 1]], buf_b, sem_b)
        pltpu.make_async_copy(tab.at[idx_s.at[j]], buf_a, sem_a).wait()
        pltpu.sync_copy(buf_a, acc.at[idx_d.at[j]], add=True)
        pltpu.async_copy(tab.at[idx_s.at[j + 2]], buf_a, sem_a)
        pltpu.make_async_copy(tab.at[idx_s.at[j + 1]], buf_b, sem_b).wait()
        pltpu.sync_copy(buf_b, acc.at[idx_d.at[j + 1]], add=True)

    last = ERT_ALL - 2
    pltpu.async_copy(tab.at[idx_s.at[last + 1]], buf_b, sem_b)
    pltpu.make_async_copy(tab.at[idx_s.at[last]], buf_a, sem_a).wait()
    pltpu.sync_copy(buf_a, acc.at[idx_d.at[last]], add=True)
    pltpu.make_async_copy(tab.at[idx_s.at[last + 1]], buf_b, sem_b).wait()
    pltpu.sync_copy(buf_b, acc.at[idx_d.at[last + 1]], add=True)

    plsc.subcore_barrier()

    for off, nr in ((0, CHUNK), (CHUNK, RPT - CHUNK)):
        rows0 = s * RPT + off
        pltpu.sync_copy(acc.at[pl.ds(rows0, nr)], buf_a.at[pl.ds(0, nr)])
        pltpu.sync_copy(y2_hbm.at[c, pl.ds(rows0, nr)], buf_b.at[pl.ds(0, nr)])
        pltpu.sync_copy(dinv_hbm.at[pl.ds(rows0, nr)], dinv_t.at[pl.ds(0, nr)])

        @pl.loop(0, nr)
        def _(r):
            d = dinv_t[r, pl.ds(0, 16)]
            for t in range(WF // 16):
                col = pl.ds(t * 16, 16)
                buf_a[r, col] = (d * (buf_a[r, col] + buf_b[r, col])
                                 + b2_t[pl.ds(t * 16, 16)])

        pltpu.sync_copy(buf_a.at[pl.ds(0, nr)],
                        out_ref.at[pl.ds(rows0, nr), pl.ds(c * WF, WF)])

_B = 2000


def _dinv_from_cnt(cnt_blk):
    deg = cnt_blk[0, :, 0:1] + cnt_blk[1, :, 0:1] + 1.0
    return lax.rsqrt(deg)


def _tc1_body(x_ref, w1_ref, cnt_ref, y1_ref, dinv_ref):
    dinv = _dinv_from_cnt(cnt_ref)
    xw = jnp.dot(x_ref[...], w1_ref[...], preferred_element_type=jnp.float32)
    y1_ref[...] = xw * dinv
    dinv_ref[...] = jnp.broadcast_to(dinv, (dinv.shape[0], 16))


def _tc2_body(a_ref, y1_ref, cnt_ref, b1_ref, w2_ref, y2_ref):
    dinv = _dinv_from_cnt(cnt_ref)
    pre = (a_ref[0] + a_ref[1] + y1_ref[...]) * dinv + b1_ref[...]
    h = jnp.maximum(pre, 0.0)
    hw = jnp.dot(h, w2_ref[...], preferred_element_type=jnp.float32)
    y2 = hw * dinv
    y2_ref[0] = y2[:, :WF]
    y2_ref[1] = y2[:, WF:]


def _tc1(x, W1, cnt):
    return pl.pallas_call(
        _tc1_body,
        grid=(N // _B,),
        in_specs=[
            pl.BlockSpec((_B, D_IN), lambda i: (i, 0)),
            pl.BlockSpec((D_IN, D_HID), lambda i: (0, 0)),
            pl.BlockSpec((NC, _B, 16), lambda i: (0, i, 0)),
        ],
        out_specs=[
            pl.BlockSpec((_B, D_HID), lambda i: (i, 0)),
            pl.BlockSpec((_B, 16), lambda i: (i, 0)),
        ],
        out_shape=[
            jax.ShapeDtypeStruct((N, D_HID), jnp.float32),
            jax.ShapeDtypeStruct((N, 16), jnp.float32),
        ],
    )(x, W1, cnt)


def _tc2(a, y1, cnt, b1, W2):
    return pl.pallas_call(
        _tc2_body,
        grid=(N // _B,),
        in_specs=[
            pl.BlockSpec((NC, _B, D_HID), lambda i: (0, i, 0)),
            pl.BlockSpec((_B, D_HID), lambda i: (i, 0)),
            pl.BlockSpec((NC, _B, 16), lambda i: (0, i, 0)),
            pl.BlockSpec((1, D_HID), lambda i: (0, 0)),
            pl.BlockSpec((D_HID, D_OUT), lambda i: (0, 0)),
        ],
        out_specs=pl.BlockSpec((2, _B, WF), lambda i: (0, i, 0)),
        out_shape=jax.ShapeDtypeStruct((2, N, WF), jnp.float32),
    )(a, y1, cnt, b1, W2)


def kernel(x, edge_index, W1, b1, W2, b2):
    src = edge_index[0].astype(jnp.int32)
    dst = edge_index[1].astype(jnp.int32)
    src5 = src.reshape(EROWS, CHUNK)
    dst5 = dst.reshape(EROWS, CHUNK)
    src1 = src.reshape(ER1, CH1)
    dst1 = dst.reshape(ER1, CH1)
    cnt = _sc_degree(dst5)
    y1, dinv = _tc1(x, W1, cnt)
    a = _sc_scatter_l1(src1, dst1, y1)
    y2 = _tc2(a, y1, cnt, b1.reshape(1, D_HID), W2)
    return _sc_scatter_l2f(src5, dst5, y2, dinv, b2.reshape(1, D_OUT))
